# baseline (speedup 1.0000x reference)
import jax
import jax.numpy as jnp
from jax import lax
from jax.experimental import pallas as pl
from jax.experimental.pallas import tpu as pltpu
from jax.experimental.pallas import tpu_sc as plsc

N = 10000
E = 320000
D = 128
K = 2
ALPHA = 0.5

NC = 2
NS = 16
L = 16
NW = NC * NS
EPT = E // NW
NP = 10240
SLC = NP // NS

NPH = 5
PH_BASE = (0, 2040, 4080, 6120, 8160)
PH_SIZE = (2040, 2040, 2040, 2040, 1840)
ACC_R = 2048
BBP = 16
ARENA = 10752
BBD = 16

_mesh = plsc.VectorSubcoreMesh(
    core_axis_name="c", subcore_axis_name="s", num_cores=NC, num_subcores=NS)
_sc_params = pltpu.CompilerParams(needs_layout_passes=False)


def _phase_mask(d16, p):
    if p == 0:
        return d16 < PH_BASE[1]
    if p == NPH - 1:
        return d16 >= PH_BASE[NPH - 1]
    return (d16 >= PH_BASE[p]) & (d16 < PH_BASE[p] + PH_SIZE[p])


def _prep_body(src_hbm, dst_hbm, degp_hbm, sl_hbm, dl_hbm, cnt_hbm,
               srcv, dstv, sarena, darena, onesv, res, cntv, shared):
    c = lax.axis_index("c")
    s = lax.axis_index("s")
    wid = c * NS + s
    pltpu.sync_copy(src_hbm.at[pl.ds(wid * EPT, EPT)], srcv)
    pltpu.sync_copy(dst_hbm.at[pl.ds(wid * EPT, EPT)], dstv)

    zeros = jnp.zeros((L,), jnp.float32)
    ones = jnp.ones((L,), jnp.float32)

    @pl.loop(0, SLC // L)
    def _zres(i):
        res[pl.ds(i * L, L)] = zeros

    @pl.loop(0, BBD // L)
    def _zone(i):
        onesv[pl.ds(i * L, L)] = ones

    pltpu.sync_copy(res, shared.at[pl.ds(s * SLC, SLC)])
    plsc.subcore_barrier()

    @pl.loop(0, EPT // BBD)
    def _deg(j):
        pltpu.sync_copy(onesv, shared.at[dstv.at[pl.ds(j * BBD, BBD)]],
                        add=True)

    plsc.subcore_barrier()
    pltpu.sync_copy(shared.at[pl.ds(s * SLC, SLC)], res)
    pltpu.sync_copy(res, degp_hbm.at[pl.ds(c * NP + s * SLC, SLC)])

    def _cnt(i, carry):
        d16 = dstv[pl.ds(i * L, L)]
        return tuple(
            carry[p] + jnp.max(plsc.all_reduce_population_count(
                _phase_mask(d16, p)))
            for p in range(NPH))

    cnts = pl.loop(0, EPT // L, init_carry=(0,) * NPH)(_cnt)

    zi16 = jnp.zeros((L,), jnp.int32)
    bases = []
    b = 0
    for p in range(NPH):
        bases.append(b)
        b = b + ((cnts[p] + BBP - 1) // BBP) * BBP
        trash16 = jnp.full((L,), PH_SIZE[p], jnp.int32)
        for k in range(BBP // L):
            sarena[pl.ds(bases[p] + cnts[p] + k * L, L)] = zi16
            darena[pl.ds(bases[p] + cnts[p] + k * L, L)] = trash16

    def _fill(i, carry):
        s16 = srcv[pl.ds(i * L, L)]
        d16 = dstv[pl.ds(i * L, L)]
        new = []
        for p in range(NPH):
            m = _phase_mask(d16, p)
            o = carry[p]
            plsc.store_compressed(sarena.at[pl.ds(o, L)], s16, mask=m)
            plsc.store_compressed(darena.at[pl.ds(o, L)],
                                  d16 - PH_BASE[p], mask=m)
            new.append(o + jnp.max(plsc.all_reduce_population_count(m)))
        return tuple(new)

    pl.loop(0, EPT // L, init_carry=tuple(bases))(_fill)

    pltpu.sync_copy(sarena, sl_hbm.at[pl.ds(wid * ARENA, ARENA)])
    pltpu.sync_copy(darena, dl_hbm.at[pl.ds(wid * ARENA, ARENA)])

    iota16 = lax.iota(jnp.int32, L)
    cv = jnp.zeros((L,), jnp.int32)
    for p in range(NPH):
        nb = (cnts[p] + BBP - 1) // BBP
        cv = jnp.where(iota16 == p, nb, cv)
    cntv[pl.ds(0, L)] = cv
    pltpu.sync_copy(cntv, cnt_hbm.at[pl.ds(wid * L, L)])


_prep_call = pl.kernel(
    _prep_body,
    out_type=[
        jax.ShapeDtypeStruct((NC * NP,), jnp.float32),
        jax.ShapeDtypeStruct((NW * ARENA,), jnp.int32),
        jax.ShapeDtypeStruct((NW * ARENA,), jnp.int32),
        jax.ShapeDtypeStruct((NW * L,), jnp.int32),
    ],
    mesh=_mesh,
    compiler_params=_sc_params,
    scratch_types=[
        pltpu.VMEM((EPT,), jnp.int32),
        pltpu.VMEM((EPT,), jnp.int32),
        pltpu.VMEM((ARENA,), jnp.int32),
        pltpu.VMEM((ARENA,), jnp.int32),
        pltpu.VMEM((BBD,), jnp.float32),
        pltpu.VMEM((SLC,), jnp.float32),
        pltpu.VMEM((L,), jnp.int32),
        pltpu.VMEM_SHARED((NP,), jnp.float32),
    ],
)


NBUF = 8


def _prop_body(g_hbm, sl_hbm, dl_hbm, cnt_hbm, out_hbm,
               sarena, darena, cntv,
               buf0, buf1, buf2, buf3, buf4, buf5, buf6, buf7, zbuf, acc,
               gs0, gs1, gs2, gs3, gs4, gs5, gs6, gs7,
               ss0, ss1, ss2, ss3, ss4, ss5, ss6, ss7):
    bufs = (buf0, buf1, buf2, buf3, buf4, buf5, buf6, buf7)
    gsems = (gs0, gs1, gs2, gs3, gs4, gs5, gs6, gs7)
    ssems = (ss0, ss1, ss2, ss3, ss4, ss5, ss6, ss7)
    c = lax.axis_index("c")
    s = lax.axis_index("s")
    wid = c * NS + s
    pltpu.sync_copy(sl_hbm.at[pl.ds(wid * ARENA, ARENA)], sarena)
    pltpu.sync_copy(dl_hbm.at[pl.ds(wid * ARENA, ARENA)], darena)
    pltpu.sync_copy(cnt_hbm.at[pl.ds(wid * L, L)], cntv)

    zeros = jnp.zeros((L,), jnp.float32)

    @pl.loop(0, ACC_R // NS)
    def _zrow(i):
        for j in range(D // L):
            zbuf[i, pl.ds(j * L, L)] = zeros

    iota16 = lax.iota(jnp.int32, L)
    cvec = cntv[pl.ds(0, L)]

    nbs = [jnp.max(jnp.where(iota16 == p, cvec, 0)) for p in range(NPH)]
    pbases = []
    base = 0
    for p in range(NPH):
        pbases.append(pl.multiple_of(base, BBP))
        base = base + nbs[p] * BBP

    def _goff(p, j):
        return pl.multiple_of(pbases[p] + j * BBP, BBP)

    def _gather(p, j, r):
        pltpu.async_copy(g_hbm.at[sarena.at[pl.ds(_goff(p, j), BBP)]],
                         bufs[r], gsems[r])

    def _gather_wait(p, j, r):
        pltpu.make_async_copy(
            g_hbm.at[sarena.at[pl.ds(_goff(p, j), BBP)]],
            bufs[r], gsems[r]).wait()

    def _scatter(p, j, r):
        pltpu.async_copy(bufs[r], acc.at[darena.at[pl.ds(_goff(p, j), BBP)]],
                         ssems[r], add=True)

    def _scatter_wait(p, j, r):
        pltpu.make_async_copy(
            bufs[r], acc.at[darena.at[pl.ds(_goff(p, j), BBP)]],
            ssems[r]).wait()

    def _prime(p):
        for r in range(NBUF - 1):
            @pl.when(r < nbs[p])
            def _pr():
                _gather(p, r, r)

    _prime(0)
    for p in range(NPH):
        nb = nbs[p]
        pltpu.sync_copy(zbuf, acc.at[pl.ds(s * (ACC_R // NS), ACC_R // NS)])
        plsc.subcore_barrier()

        @pl.loop(0, (nb + NBUF - 1) // NBUF)
        def _quads(q):
            for r in range(NBUF):
                j = q * NBUF + r
                r3 = (r + NBUF - 1) % NBUF

                @pl.when(j < nb)
                def _do():
                    _gather_wait(p, j, r)
                    _scatter(p, j, r)

                @pl.when((j >= 1) & (j + NBUF - 1 < nb))
                def _free():
                    _scatter_wait(p, j - 1, r3)

                @pl.when(j + NBUF - 1 < nb)
                def _ahead():
                    _gather(p, j + NBUF - 1, r3)

        for r in range(NBUF):
            @pl.when(r < nb)
            def _drain():
                pltpu.make_async_copy(
                    bufs[r], acc.at[darena.at[pl.ds(pbases[p], BBP)]],
                    ssems[r]).wait()

        if p + 1 < NPH:
            _prime(p + 1)
        plsc.subcore_barrier()
        if p < NPH - 1:
            pltpu.sync_copy(
                acc.at[pl.ds(s * 128, 120)],
                out_hbm.at[c, pl.ds(PH_BASE[p] + s * 128, 120)])

            @pl.when(s != NS - 1)
            def _tail_a():
                pltpu.sync_copy(
                    acc.at[pl.ds(s * 128 + 120, 8)],
                    out_hbm.at[c, pl.ds(PH_BASE[p] + s * 128 + 120, 8)])
        else:
            pltpu.sync_copy(
                acc.at[pl.ds(s * 112, 112)],
                out_hbm.at[c, pl.ds(PH_BASE[p] + s * 112, 112)])

            @pl.when(s == NS - 1)
            def _tail_b():
                pltpu.sync_copy(
                    acc.at[pl.ds(1792, 48)],
                    out_hbm.at[c, pl.ds(PH_BASE[p] + 1792, 48)])


_prop_call = pl.kernel(
    _prop_body,
    out_type=jax.ShapeDtypeStruct((NC, N, D), jnp.float32),
    mesh=_mesh,
    compiler_params=_sc_params,
    scratch_types=[
        pltpu.VMEM((ARENA,), jnp.int32),
        pltpu.VMEM((ARENA,), jnp.int32),
        pltpu.VMEM((L,), jnp.int32),
    ] + [pltpu.VMEM((BBP, D), jnp.float32)] * 8 + [
        pltpu.VMEM((ACC_R // NS, D), jnp.float32),
        pltpu.VMEM_SHARED((ACC_R, D), jnp.float32),
    ] + [pltpu.SemaphoreType.DMA] * (2 * NBUF),
)


_RN = 2000


def _norm_body(dp_ref, init_ref, norm_ref, ri_ref, g_ref):
    d = dp_ref[0] + dp_ref[1]
    nrm = lax.rsqrt(d + 1.0)
    init = init_ref[...]
    norm_ref[...] = nrm
    ri_ref[...] = init * (nrm * nrm)
    g_ref[...] = init * nrm


_norm_call = pl.pallas_call(
    _norm_body,
    grid=(N // _RN,),
    in_specs=[
        pl.BlockSpec((2, _RN, 1), lambda i: (0, i, 0)),
        pl.BlockSpec((_RN, D), lambda i: (i, 0)),
    ],
    out_specs=[
        pl.BlockSpec((_RN, 1), lambda i: (i, 0)),
        pl.BlockSpec((_RN, D), lambda i: (i, 0)),
        pl.BlockSpec((_RN, D), lambda i: (i, 0)),
    ],
    out_shape=[
        jax.ShapeDtypeStruct((N, 1), jnp.float32),
        jax.ShapeDtypeStruct((N, D), jnp.float32),
        jax.ShapeDtypeStruct((N, D), jnp.float32),
    ],
)


def _combine_body(sp_ref, norm_ref, ri_ref, hp_ref, h_ref, g_ref):
    svec = sp_ref[0] + sp_ref[1]
    nrm = norm_ref[...]
    h = ALPHA * (svec * nrm) + ALPHA * ri_ref[...] + (1.0 - ALPHA) * hp_ref[...]
    h_ref[...] = h
    g_ref[...] = h * nrm


_combine_call = pl.pallas_call(
    _combine_body,
    grid=(N // _RN,),
    in_specs=[
        pl.BlockSpec((2, _RN, D), lambda i: (0, i, 0)),
        pl.BlockSpec((_RN, 1), lambda i: (i, 0)),
        pl.BlockSpec((_RN, D), lambda i: (i, 0)),
        pl.BlockSpec((_RN, D), lambda i: (i, 0)),
    ],
    out_specs=[
        pl.BlockSpec((_RN, D), lambda i: (i, 0)),
        pl.BlockSpec((_RN, D), lambda i: (i, 0)),
    ],
    out_shape=[
        jax.ShapeDtypeStruct((N, D), jnp.float32),
        jax.ShapeDtypeStruct((N, D), jnp.float32),
    ],
)


def kernel(features, initial_features, edge_index):
    del features
    src = edge_index[0]
    dst = edge_index[1]

    degp, sl, dl, cnt = _prep_call(src, dst)
    dp = degp.reshape(NC, NP)[:, :N].reshape(NC, N, 1)
    norm, ri, g0 = _norm_call(dp, initial_features)

    def _step(carry, _):
        h_pre, g = carry
        sp = _prop_call(g, sl, dl, cnt)
        h, gn = _combine_call(sp, norm, ri, h_pre)
        return (h, gn), None

    (h, _), _ = lax.scan(_step, (initial_features, g0), None, length=K)
    return h


# speedup vs baseline: 1.0780x; 1.0780x over previous
import jax
import jax.numpy as jnp
from jax import lax
from jax.experimental import pallas as pl
from jax.experimental.pallas import tpu as pltpu
from jax.experimental.pallas import tpu_sc as plsc

N = 10000
E = 320000
D = 128
K = 2
ALPHA = 0.5

NC = 2
NS = 16
L = 16
NW = NC * NS
EPT = E // NW
NP = 10240
SLC = NP // NS

NPH = 5
PH_BASE = (0, 2040, 4080, 6120, 8160)
PH_SIZE = (2040, 2040, 2040, 2040, 1840)
ACC_R = 2048
BBP = 16
ARENA = 10752
BBD = 80

_mesh = plsc.VectorSubcoreMesh(
    core_axis_name="c", subcore_axis_name="s", num_cores=NC, num_subcores=NS)
_sc_params = pltpu.CompilerParams(needs_layout_passes=False)


def _phase_mask(d16, p):
    if p == 0:
        return d16 < PH_BASE[1]
    if p == NPH - 1:
        return d16 >= PH_BASE[NPH - 1]
    return (d16 >= PH_BASE[p]) & (d16 < PH_BASE[p] + PH_SIZE[p])


def _prep_body(src_hbm, dst_hbm, degp_hbm, sl_hbm, dl_hbm, cnt_hbm,
               srcv, dstv, sarena, darena, onesv, res, cntv, shared):
    c = lax.axis_index("c")
    s = lax.axis_index("s")
    wid = c * NS + s
    pltpu.sync_copy(src_hbm.at[pl.ds(wid * EPT, EPT)], srcv)
    pltpu.sync_copy(dst_hbm.at[pl.ds(wid * EPT, EPT)], dstv)

    zeros = jnp.zeros((L,), jnp.float32)
    ones = jnp.ones((L,), jnp.float32)

    @pl.loop(0, SLC // L)
    def _zres(i):
        res[pl.ds(i * L, L)] = zeros

    @pl.loop(0, BBD // L)
    def _zone(i):
        onesv[pl.ds(i * L, L)] = ones

    pltpu.sync_copy(res, shared.at[pl.ds(s * SLC, SLC)])
    plsc.subcore_barrier()

    @pl.loop(0, EPT // BBD)
    def _deg(j):
        pltpu.sync_copy(onesv, shared.at[dstv.at[pl.ds(j * BBD, BBD)]],
                        add=True)

    plsc.subcore_barrier()
    pltpu.sync_copy(shared.at[pl.ds(s * SLC, SLC)], res)
    pltpu.sync_copy(res, degp_hbm.at[pl.ds(c * NP + s * SLC, SLC)])

    def _cnt(i, carry):
        d16 = dstv[pl.ds(i * L, L)]
        return tuple(
            carry[p] + jnp.max(plsc.all_reduce_population_count(
                _phase_mask(d16, p)))
            for p in range(NPH))

    cnts = pl.loop(0, EPT // L, init_carry=(0,) * NPH)(_cnt)

    zi16 = jnp.zeros((L,), jnp.int32)
    bases = []
    b = 0
    for p in range(NPH):
        bases.append(b)
        b = b + ((cnts[p] + BBP - 1) // BBP) * BBP
        trash16 = jnp.full((L,), PH_SIZE[p], jnp.int32)
        for k in range(BBP // L):
            sarena[pl.ds(bases[p] + cnts[p] + k * L, L)] = zi16
            darena[pl.ds(bases[p] + cnts[p] + k * L, L)] = trash16

    def _fill(i, carry):
        s16 = srcv[pl.ds(i * L, L)]
        d16 = dstv[pl.ds(i * L, L)]
        new = []
        for p in range(NPH):
            m = _phase_mask(d16, p)
            o = carry[p]
            plsc.store_compressed(sarena.at[pl.ds(o, L)], s16, mask=m)
            plsc.store_compressed(darena.at[pl.ds(o, L)],
                                  d16 - PH_BASE[p], mask=m)
            new.append(o + jnp.max(plsc.all_reduce_population_count(m)))
        return tuple(new)

    pl.loop(0, EPT // L, init_carry=tuple(bases))(_fill)

    pltpu.sync_copy(sarena, sl_hbm.at[pl.ds(wid * ARENA, ARENA)])
    pltpu.sync_copy(darena, dl_hbm.at[pl.ds(wid * ARENA, ARENA)])

    iota16 = lax.iota(jnp.int32, L)
    cv = jnp.zeros((L,), jnp.int32)
    for p in range(NPH):
        nb = (cnts[p] + BBP - 1) // BBP
        cv = jnp.where(iota16 == p, nb, cv)
    cntv[pl.ds(0, L)] = cv
    pltpu.sync_copy(cntv, cnt_hbm.at[pl.ds(wid * L, L)])


_prep_call = pl.kernel(
    _prep_body,
    out_type=[
        jax.ShapeDtypeStruct((NC * NP,), jnp.float32),
        jax.ShapeDtypeStruct((NW * ARENA,), jnp.int32),
        jax.ShapeDtypeStruct((NW * ARENA,), jnp.int32),
        jax.ShapeDtypeStruct((NW * L,), jnp.int32),
    ],
    mesh=_mesh,
    compiler_params=_sc_params,
    scratch_types=[
        pltpu.VMEM((EPT,), jnp.int32),
        pltpu.VMEM((EPT,), jnp.int32),
        pltpu.VMEM((ARENA,), jnp.int32),
        pltpu.VMEM((ARENA,), jnp.int32),
        pltpu.VMEM((BBD,), jnp.float32),
        pltpu.VMEM((SLC,), jnp.float32),
        pltpu.VMEM((L,), jnp.int32),
        pltpu.VMEM_SHARED((NP,), jnp.float32),
    ],
)


NBUF = 8


def _prop_body(g_hbm, sl_hbm, dl_hbm, cnt_hbm, out_hbm,
               sarena, darena, cntv,
               buf0, buf1, buf2, buf3, buf4, buf5, buf6, buf7, zbuf, acc,
               gs0, gs1, gs2, gs3, gs4, gs5, gs6, gs7,
               ss0, ss1, ss2, ss3, ss4, ss5, ss6, ss7):
    bufs = (buf0, buf1, buf2, buf3, buf4, buf5, buf6, buf7)
    gsems = (gs0, gs1, gs2, gs3, gs4, gs5, gs6, gs7)
    ssems = (ss0, ss1, ss2, ss3, ss4, ss5, ss6, ss7)
    c = lax.axis_index("c")
    s = lax.axis_index("s")
    wid = c * NS + s
    pltpu.sync_copy(sl_hbm.at[pl.ds(wid * ARENA, ARENA)], sarena)
    pltpu.sync_copy(dl_hbm.at[pl.ds(wid * ARENA, ARENA)], darena)
    pltpu.sync_copy(cnt_hbm.at[pl.ds(wid * L, L)], cntv)

    zeros = jnp.zeros((L,), jnp.float32)

    @pl.loop(0, ACC_R // NS)
    def _zrow(i):
        for j in range(D // L):
            zbuf[i, pl.ds(j * L, L)] = zeros

    iota16 = lax.iota(jnp.int32, L)
    cvec = cntv[pl.ds(0, L)]

    nbs = [jnp.max(jnp.where(iota16 == p, cvec, 0)) for p in range(NPH)]
    pbases = []
    base = 0
    for p in range(NPH):
        pbases.append(pl.multiple_of(base, BBP))
        base = base + nbs[p] * BBP

    def _goff(p, j):
        return pl.multiple_of(pbases[p] + j * BBP, BBP)

    def _gather(p, j, r):
        pltpu.async_copy(g_hbm.at[sarena.at[pl.ds(_goff(p, j), BBP)]],
                         bufs[r], gsems[r])

    def _gather_wait(p, j, r):
        pltpu.make_async_copy(
            g_hbm.at[sarena.at[pl.ds(_goff(p, j), BBP)]],
            bufs[r], gsems[r]).wait()

    def _scatter(p, j, r):
        pltpu.async_copy(bufs[r], acc.at[darena.at[pl.ds(_goff(p, j), BBP)]],
                         ssems[r], add=True)

    def _scatter_wait(p, j, r):
        pltpu.make_async_copy(
            bufs[r], acc.at[darena.at[pl.ds(_goff(p, j), BBP)]],
            ssems[r]).wait()

    def _prime(p):
        for r in range(NBUF - 1):
            @pl.when(r < nbs[p])
            def _pr():
                _gather(p, r, r)

    _prime(0)
    for p in range(NPH):
        nb = nbs[p]
        pltpu.sync_copy(zbuf, acc.at[pl.ds(s * (ACC_R // NS), ACC_R // NS)])
        plsc.subcore_barrier()

        @pl.loop(0, (nb + NBUF - 1) // NBUF)
        def _quads(q):
            for r in range(NBUF):
                j = q * NBUF + r
                r3 = (r + NBUF - 1) % NBUF

                @pl.when(j < nb)
                def _do():
                    _gather_wait(p, j, r)
                    _scatter(p, j, r)

                @pl.when((j >= 1) & (j + NBUF - 1 < nb))
                def _free():
                    _scatter_wait(p, j - 1, r3)

                @pl.when(j + NBUF - 1 < nb)
                def _ahead():
                    _gather(p, j + NBUF - 1, r3)

        for r in range(NBUF):
            @pl.when(r < nb)
            def _drain():
                pltpu.make_async_copy(
                    bufs[r], acc.at[darena.at[pl.ds(pbases[p], BBP)]],
                    ssems[r]).wait()

        if p + 1 < NPH:
            _prime(p + 1)
        plsc.subcore_barrier()
        if p < NPH - 1:
            pltpu.sync_copy(
                acc.at[pl.ds(s * 128, 120)],
                out_hbm.at[c, pl.ds(PH_BASE[p] + s * 128, 120)])

            @pl.when(s != NS - 1)
            def _tail_a():
                pltpu.sync_copy(
                    acc.at[pl.ds(s * 128 + 120, 8)],
                    out_hbm.at[c, pl.ds(PH_BASE[p] + s * 128 + 120, 8)])
        else:
            pltpu.sync_copy(
                acc.at[pl.ds(s * 112, 112)],
                out_hbm.at[c, pl.ds(PH_BASE[p] + s * 112, 112)])

            @pl.when(s == NS - 1)
            def _tail_b():
                pltpu.sync_copy(
                    acc.at[pl.ds(1792, 48)],
                    out_hbm.at[c, pl.ds(PH_BASE[p] + 1792, 48)])


_prop_call = pl.kernel(
    _prop_body,
    out_type=jax.ShapeDtypeStruct((NC, N, D), jnp.float32),
    mesh=_mesh,
    compiler_params=_sc_params,
    scratch_types=[
        pltpu.VMEM((ARENA,), jnp.int32),
        pltpu.VMEM((ARENA,), jnp.int32),
        pltpu.VMEM((L,), jnp.int32),
    ] + [pltpu.VMEM((BBP, D), jnp.float32)] * 8 + [
        pltpu.VMEM((ACC_R // NS, D), jnp.float32),
        pltpu.VMEM_SHARED((ACC_R, D), jnp.float32),
    ] + [pltpu.SemaphoreType.DMA] * (2 * NBUF),
)


_RN = 2000


def _norm_body(dp_ref, init_ref, norm_ref, ri_ref, g_ref):
    d = dp_ref[0] + dp_ref[1]
    nrm = lax.rsqrt(d + 1.0)
    init = init_ref[...]
    norm_ref[...] = nrm
    ri_ref[...] = init * (nrm * nrm)
    g_ref[...] = init * nrm


_norm_call = pl.pallas_call(
    _norm_body,
    grid=(N // _RN,),
    in_specs=[
        pl.BlockSpec((2, _RN, 1), lambda i: (0, i, 0)),
        pl.BlockSpec((_RN, D), lambda i: (i, 0)),
    ],
    out_specs=[
        pl.BlockSpec((_RN, 1), lambda i: (i, 0)),
        pl.BlockSpec((_RN, D), lambda i: (i, 0)),
        pl.BlockSpec((_RN, D), lambda i: (i, 0)),
    ],
    out_shape=[
        jax.ShapeDtypeStruct((N, 1), jnp.float32),
        jax.ShapeDtypeStruct((N, D), jnp.float32),
        jax.ShapeDtypeStruct((N, D), jnp.float32),
    ],
)


def _combine_body(sp_ref, norm_ref, ri_ref, hp_ref, h_ref, g_ref):
    svec = sp_ref[0] + sp_ref[1]
    nrm = norm_ref[...]
    h = ALPHA * (svec * nrm) + ALPHA * ri_ref[...] + (1.0 - ALPHA) * hp_ref[...]
    h_ref[...] = h
    g_ref[...] = h * nrm


_combine_call = pl.pallas_call(
    _combine_body,
    grid=(N // _RN,),
    in_specs=[
        pl.BlockSpec((2, _RN, D), lambda i: (0, i, 0)),
        pl.BlockSpec((_RN, 1), lambda i: (i, 0)),
        pl.BlockSpec((_RN, D), lambda i: (i, 0)),
        pl.BlockSpec((_RN, D), lambda i: (i, 0)),
    ],
    out_specs=[
        pl.BlockSpec((_RN, D), lambda i: (i, 0)),
        pl.BlockSpec((_RN, D), lambda i: (i, 0)),
    ],
    out_shape=[
        jax.ShapeDtypeStruct((N, D), jnp.float32),
        jax.ShapeDtypeStruct((N, D), jnp.float32),
    ],
)


def kernel(features, initial_features, edge_index):
    del features
    src = edge_index[0]
    dst = edge_index[1]

    degp, sl, dl, cnt = _prep_call(src, dst)
    dp = degp.reshape(NC, NP)[:, :N].reshape(NC, N, 1)
    norm, ri, g0 = _norm_call(dp, initial_features)

    def _step(carry, _):
        h_pre, g = carry
        sp = _prop_call(g, sl, dl, cnt)
        h, gn = _combine_call(sp, norm, ri, h_pre)
        return (h, gn), None

    (h, _), _ = lax.scan(_step, (initial_features, g0), None, length=K)
    return h
